# Initial kernel scaffold; baseline (speedup 1.0000x reference)
import jax
import jax.numpy as jnp
from jax.experimental import pallas as pl


def kernel(x, edge_index, W, b, prelu_a):
    raise NotImplementedError("write your pallas kernel here")


# speedup vs baseline: 14.0627x; 14.0627x over previous
import functools

import jax
import jax.numpy as jnp
from jax import lax
from jax.experimental import pallas as pl
from jax.experimental.pallas import tpu as pltpu
from jax.experimental.pallas import tpu_sc as plsc

N = 10000
D = 128
E = 320000

NC = 2
NS = 16
NW = NC * NS

K = 128
CPT = 79
E_PAD = NW * CPT * K
EPT = CPT * K

N_PAD = 10240
ROWS16 = N_PAD // 16
RB = ROWS16 // NS

_mesh = plsc.VectorSubcoreMesh(core_axis_name="c", subcore_axis_name="s")
_sc_params = pltpu.CompilerParams(needs_layout_passes=False)


@functools.partial(
    pl.kernel,
    out_type=jax.ShapeDtypeStruct((NW, N_PAD), jnp.float32),
    mesh=_mesh,
    compiler_params=_sc_params,
    scratch_types=[
        pltpu.VMEM((EPT,), jnp.int32),
        pltpu.VMEM((N_PAD,), jnp.float32),
    ],
)
def _deg_kernel(dst_hbm, deg_out, dstv, hist):
    c = lax.axis_index("c")
    s = lax.axis_index("s")
    w = c * NS + s
    pltpu.sync_copy(dst_hbm.at[w], dstv)

    zero16 = jnp.zeros((16,), jnp.float32)

    def _zero(i, carry):
        hist[pl.ds(i * 16, 16)] = zero16
        return carry

    lax.fori_loop(0, N_PAD // 16, _zero, 0)

    ones16 = jnp.ones((16,), jnp.float32)

    def _count(i, carry):
        idx = dstv[pl.ds(i * 16, 16)]
        plsc.addupdate_scatter(hist, [idx], ones16)
        return carry

    lax.fori_loop(0, EPT // 16, _count, 0)

    pltpu.sync_copy(hist, deg_out.at[w])


@functools.partial(
    pl.kernel,
    out_type=jax.ShapeDtypeStruct((NC, N_PAD, D), jnp.float32),
    mesh=_mesh,
    compiler_params=_sc_params,
    scratch_types=[
        pltpu.VMEM((K,), jnp.int32),
        pltpu.VMEM((K,), jnp.int32),
        pltpu.VMEM((K, D), jnp.float32),
        pltpu.VMEM((16, D), jnp.float32),
        pltpu.SemaphoreType.DMA,
        pltpu.VMEM_SHARED((N_PAD, D), jnp.float32),
    ],
)
def _agg_kernel(src_hbm, dst_hbm, xs_hbm, out_hbm, sidx, didx, rows, ztile,
                sem, aggsh):
    c = lax.axis_index("c")
    s = lax.axis_index("s")
    w = c * NS + s

    zero16 = jnp.zeros((16,), jnp.float32)

    def _zero(t, carry):
        ztile[t // 8, pl.ds((t % 8) * 16, 16)] = zero16
        return carry

    lax.fori_loop(0, 128, _zero, 0)

    rows_per_tile = N_PAD // NS

    def _init(j, carry):
        pltpu.sync_copy(ztile, aggsh.at[pl.ds(s * rows_per_tile + j * 16, 16)])
        return carry

    lax.fori_loop(0, rows_per_tile // 16, _init, 0)
    plsc.subcore_barrier()

    def _edge(t, carry):
        pltpu.sync_copy(src_hbm.at[w, t], sidx)
        pltpu.sync_copy(dst_hbm.at[w, t], didx)
        pltpu.async_copy(xs_hbm.at[sidx], rows, sem).wait()
        pltpu.sync_copy(rows, aggsh.at[didx], add=True)
        return carry

    lax.fori_loop(0, CPT, _edge, 0)
    plsc.subcore_barrier()

    pltpu.sync_copy(
        aggsh.at[pl.ds(s * rows_per_tile, rows_per_tile)],
        out_hbm.at[c, pl.ds(s * rows_per_tile, rows_per_tile)],
    )


def _scale_body(x_ref, h_ref, xs_ref, dinv_ref):
    deg_row = jnp.sum(h_ref[...], axis=0, keepdims=True) + 1.0
    dinv_row = lax.rsqrt(deg_row)
    r = lax.broadcasted_iota(jnp.int32, (128, 128), 0)
    col = lax.broadcasted_iota(jnp.int32, (128, 128), 1)
    diag = jnp.where(r == col, dinv_row, 0.0)
    xs_ref[...] = jnp.dot(diag, x_ref[...],
                          preferred_element_type=jnp.float32)
    dinv_ref[...] = jnp.dot(diag, jnp.ones((128, 1), jnp.float32),
                            preferred_element_type=jnp.float32)


def _scale(x_pad, hists):
    nblk = N_PAD // 128
    return pl.pallas_call(
        _scale_body,
        grid=(nblk,),
        in_specs=[
            pl.BlockSpec((128, D), lambda i: (i, 0)),
            pl.BlockSpec((NW, 128), lambda i: (0, i)),
        ],
        out_specs=[
            pl.BlockSpec((128, D), lambda i: (i, 0)),
            pl.BlockSpec((128, 1), lambda i: (i, 0)),
        ],
        out_shape=[
            jax.ShapeDtypeStruct((N_PAD, D), jnp.float32),
            jax.ShapeDtypeStruct((N_PAD, 1), jnp.float32),
        ],
    )(x_pad, hists)


def _out_body(p0_ref, p1_ref, xs_ref, dinv_ref, w_ref, b_ref, a_ref, o_ref):
    a = (p0_ref[...] + p1_ref[...] + xs_ref[...]) * dinv_ref[...]
    h = jnp.dot(a, w_ref[...], preferred_element_type=jnp.float32)
    h = h + b_ref[...]
    o_ref[...] = jnp.where(h >= 0, h, a_ref[...] * h)


def _finish(p0, p1, xs, dinv, W, b2, a2):
    nblk = N_PAD // 128
    return pl.pallas_call(
        _out_body,
        grid=(nblk,),
        in_specs=[
            pl.BlockSpec((128, D), lambda i: (i, 0)),
            pl.BlockSpec((128, D), lambda i: (i, 0)),
            pl.BlockSpec((128, D), lambda i: (i, 0)),
            pl.BlockSpec((128, 1), lambda i: (i, 0)),
            pl.BlockSpec((D, D), lambda i: (0, 0)),
            pl.BlockSpec((1, D), lambda i: (0, 0)),
            pl.BlockSpec((1, 1), lambda i: (0, 0)),
        ],
        out_specs=pl.BlockSpec((128, D), lambda i: (i, 0)),
        out_shape=jax.ShapeDtypeStruct((N_PAD, D), jnp.float32),
    )(p0, p1, xs, dinv, W, b2, a2)


def kernel(x, edge_index, W, b, prelu_a):
    src = edge_index[0]
    dst = edge_index[1]
    pad = jnp.full((E_PAD - E,), N, dtype=jnp.int32)
    src_p = jnp.concatenate([src, pad]).reshape(NW, CPT, K)
    dst_flat = jnp.concatenate([dst, pad])
    dst_a = dst_flat.reshape(NW, EPT)
    dst_c = dst_flat.reshape(NW, CPT, K)

    x_pad = jnp.pad(x, ((0, N_PAD - N), (0, 0)))

    hists = _deg_kernel(dst_a)
    xs, dinv = _scale(x_pad, hists)

    agg_parts = _agg_kernel(src_p, dst_c, xs)

    out = _finish(agg_parts[0], agg_parts[1], xs, dinv, W,
                  b.reshape(1, D), prelu_a.reshape(1, 1))
    return out[:N]
